# baseline (speedup 1.0000x reference)
import functools

import jax
import jax.numpy as jnp
import numpy as np
from jax import lax
from jax.experimental import pallas as pl
from jax.experimental.pallas import tpu as pltpu
from jax.experimental.pallas import tpu_sc as plsc

N_ELEMENTS = 1000000
DIM = 32
MAX_LEN = 200
B = 4096
L = 200

NC = 2
NS = 16
NW = NC * NS

K1 = 4096
Q1 = K1 // 4
NBLK1 = 245
NPAD = NBLK1 * K1

TL = L // 8
TB = B // 128
HUNITS = TL * TB * 2
HUPW = HUNITS // NW
HROWS = 4 * 128


def _sinusoidal_pe():
    pos = np.arange(MAX_LEN, dtype=np.float32)[:, None]
    div = np.exp(np.arange(0, DIM, 2, dtype=np.float32) * (-np.log(10000.0) / DIM))
    pe = np.zeros((MAX_LEN, DIM), dtype=np.float32)
    pe[:, 0::2] = np.sin(pos * div)
    pe[:, 1::2] = np.cos(pos * div)
    return pe


_PE = _sinusoidal_pe()


def _selectors():
    e = np.zeros((4, 32, 128), dtype=np.float32)
    for k in range(4):
        for c in range(32):
            e[k, c, 32 * k + c] = 1.0
    return e


_E = _selectors()


def _tc_body(x_ref, e_ref, y_ref):
    x = x_ref[...]
    gcol = pl.program_id(0) * K1 + lax.broadcasted_iota(jnp.int32, (32, K1), 1)
    x = jnp.where(gcol < N_ELEMENTS, x, 0.0)
    acc = jnp.zeros((Q1, 128), jnp.float32)
    for k in range(4):
        xk = x[:, k * Q1:(k + 1) * Q1]
        acc = acc + lax.dot_general(
            xk, e_ref[k], (((0,), (0,)), ((), ())),
            preferred_element_type=jnp.float32)
    y_ref[...] = acc


def _relayout_table(tT, e):
    return pl.pallas_call(
        _tc_body,
        grid=(NBLK1,),
        in_specs=[
            pl.BlockSpec((32, K1), lambda i: (0, i)),
            pl.BlockSpec((4, 32, 128), lambda i: (0, 0, 0)),
        ],
        out_specs=pl.BlockSpec((Q1, 128), lambda i: (i, 0)),
        out_shape=jax.ShapeDtypeStruct((NPAD // 4, 128), jnp.float32),
    )(tT, e)


def _sc_body(table_hbm, idx4_hbm, pe_hbm, out4_hbm,
             idx_a, idx_b, m_a, m_b, pe_v, rows_a, rows_b, tbuf_a, tbuf_b,
             sem_a, sem_b, wsem_a, wsem_b):
    wid = lax.axis_index("s") * NC + lax.axis_index("c")
    base = wid * HUPW

    pltpu.sync_copy(pe_hbm, pe_v)

    iota16 = lax.iota(jnp.int32, 16)
    off_lo = (lax.shift_left(lax.shift_right_logical(iota16, 3), 10)
              + lax.shift_left(iota16 & 7, 7))
    off_hi = off_lo + 2048

    def unpack(v):
        tl = lax.shift_right_logical(v, 6)
        tb = lax.shift_right_logical(v, 1) & 31
        h = v & 1
        return tl, tb, h

    def fetch(v, idx_v, m_v, rows_v, sem):
        tl, tb, h = unpack(v)
        pltpu.sync_copy(idx4_hbm.at[tl, tb, h], idx_v)

        def m_body(q, c):
            r = idx_v[pl.ds(q * 16, 16)]
            m = ((r & -K1)
                 | lax.shift_left(r & (Q1 - 1), 2)
                 | (lax.shift_right_logical(r, 10) & 3))
            m_v[pl.ds(q * 16, 16)] = m
            return c

        lax.fori_loop(0, HROWS // 16, m_body, 0, unroll=False)
        pltpu.async_copy(table_hbm.at[m_v], rows_v, sem)

    def compute(v, rows_v, tbuf_v, wsem):
        tl, tb, h = unpack(v)
        lbase = tl * 8 + h * 4

        def t_body(li, c_):
            l = lbase + li
            lsplat = jnp.zeros((16,), jnp.int32) + l
            rbase = li * 128
            tbase = li * 4096
            for cb in range(4):
                for ci in range(8):
                    c = cb * 8 + ci
                    csplat = jnp.zeros((16,), jnp.int32) + c
                    p = plsc.load_gather(pe_v, [lsplat, csplat])
                    for u in range(8):
                        ridx = (rbase + u * 16) + iota16
                        vals = plsc.load_gather(rows_v, [ridx, csplat])
                        tbuf_v[pl.ds(tbase + cb * 1024 + ci * 128 + u * 16, 16)] = vals + p
            return c_

        lax.fori_loop(0, 4, t_body, 0, unroll=False)

        def w_body(rowid, c_):
            li = lax.shift_right_logical(rowid, 2)
            cb = rowid & 3
            l = lbase + li
            pltpu.async_copy(tbuf_v.at[pl.ds(rowid * 1024, 1024)],
                             out4_hbm.at[l, cb, tb], wsem)
            return c_

        lax.fori_loop(0, 16, w_body, 0, unroll=False)

    def drain_writes(tbuf_v, wsem):
        def wd_body(i, c_):
            pltpu.make_async_copy(tbuf_v.at[pl.ds(0, 1024)],
                                  out4_hbm.at[0, 0, 0], wsem).wait()
            return c_

        lax.fori_loop(0, 16, wd_body, 0, unroll=False)

    def drain_gather(rows_v, sem):
        pltpu.make_async_copy(table_hbm.at[pl.ds(0, HROWS)], rows_v, sem).wait()

    fetch(base, idx_a, m_a, rows_a, sem_a)

    def pair_body(n, carry):
        v0 = base + 2 * n
        v1 = v0 + 1
        vn = jnp.minimum(v0 + 2, HUNITS - 1)

        drain_gather(rows_a, sem_a)
        fetch(v1, idx_b, m_b, rows_b, sem_b)

        @pl.when(n > 0)
        def _():
            drain_writes(tbuf_a, wsem_a)

        compute(v0, rows_a, tbuf_a, wsem_a)

        drain_gather(rows_b, sem_b)
        fetch(vn, idx_a, m_a, rows_a, sem_a)

        @pl.when(n > 0)
        def _():
            drain_writes(tbuf_b, wsem_b)

        compute(v1, rows_b, tbuf_b, wsem_b)
        return carry

    lax.fori_loop(0, HUPW // 2, pair_body, 0, unroll=False)

    drain_gather(rows_a, sem_a)
    drain_writes(tbuf_a, wsem_a)
    drain_writes(tbuf_b, wsem_b)


@jax.jit
def _run(kb_ids_seq, key_emb_table):
    table_lin = _relayout_table(key_emb_table.T, jnp.asarray(_E)).reshape(NPAD, DIM)
    idx4 = kb_ids_seq.T.reshape(TL, 8, TB, 128).transpose(0, 2, 1, 3).reshape(TL, TB, 2, 512)
    mesh = plsc.VectorSubcoreMesh(core_axis_name="c", subcore_axis_name="s")
    f = pl.kernel(
        _sc_body,
        out_type=jax.ShapeDtypeStruct((L, 4, TB, 1024), jnp.float32),
        mesh=mesh,
        scratch_types=[
            pltpu.VMEM((HROWS,), jnp.int32),
            pltpu.VMEM((HROWS,), jnp.int32),
            pltpu.VMEM((HROWS,), jnp.int32),
            pltpu.VMEM((HROWS,), jnp.int32),
            pltpu.VMEM((MAX_LEN, DIM), jnp.float32),
            pltpu.VMEM((HROWS, DIM), jnp.float32),
            pltpu.VMEM((HROWS, DIM), jnp.float32),
            pltpu.VMEM((16 * 1024,), jnp.float32),
            pltpu.VMEM((16 * 1024,), jnp.float32),
            pltpu.SemaphoreType.DMA,
            pltpu.SemaphoreType.DMA,
            pltpu.SemaphoreType.DMA,
            pltpu.SemaphoreType.DMA,
        ],
        compiler_params=pltpu.CompilerParams(
            use_tc_tiling_on_sc=False, needs_layout_passes=False),
    )
    out4 = f(table_lin, idx4, jnp.asarray(_PE))
    out = (out4.reshape(L, 4, TB, 8, 128)
           .transpose(2, 4, 0, 1, 3)
           .reshape(B, L, DIM))
    return out


def kernel(kb_ids_seq, key_emb_table):
    return _run(kb_ids_seq, key_emb_table)


# speedup vs baseline: 1.2032x; 1.2032x over previous
import functools

import jax
import jax.numpy as jnp
import numpy as np
from jax import lax
from jax.experimental import pallas as pl
from jax.experimental.pallas import tpu as pltpu
from jax.experimental.pallas import tpu_sc as plsc

N_ELEMENTS = 1000000
DIM = 32
MAX_LEN = 200
B = 4096
L = 200

NC = 2
NS = 16
NW = NC * NS

K1 = 4096
Q1 = K1 // 4
NBLK1 = 245
NPAD = NBLK1 * K1

TL = L // 8
TB = B // 128
HUNITS = TL * TB * 2
HUPW = HUNITS // NW
HROWS = 4 * 128


def _sinusoidal_pe():
    pos = np.arange(MAX_LEN, dtype=np.float32)[:, None]
    div = np.exp(np.arange(0, DIM, 2, dtype=np.float32) * (-np.log(10000.0) / DIM))
    pe = np.zeros((MAX_LEN, DIM), dtype=np.float32)
    pe[:, 0::2] = np.sin(pos * div)
    pe[:, 1::2] = np.cos(pos * div)
    return pe


_PE = _sinusoidal_pe()


def _selectors():
    e = np.zeros((4, 32, 128), dtype=np.float32)
    for k in range(4):
        for c in range(32):
            e[k, c, 32 * k + c] = 1.0
    return e


_E = _selectors()


def _tc_body(x_ref, e_ref, y_ref):
    def dots(x):
        acc = jnp.zeros((Q1, 128), jnp.float32)
        for k in range(4):
            xk = x[:, k * Q1:(k + 1) * Q1]
            acc = acc + lax.dot_general(
                xk, e_ref[k], (((0,), (0,)), ((), ())),
                preferred_element_type=jnp.float32)
        return acc

    pid = pl.program_id(0)

    @pl.when(pid != NBLK1 - 1)
    def _():
        y_ref[...] = dots(x_ref[...])

    @pl.when(pid == NBLK1 - 1)
    def _():
        gcol = pid * K1 + lax.broadcasted_iota(jnp.int32, (32, K1), 1)
        y_ref[...] = dots(jnp.where(gcol < N_ELEMENTS, x_ref[...], 0.0))


def _relayout_table(tT, e):
    return pl.pallas_call(
        _tc_body,
        grid=(NBLK1,),
        in_specs=[
            pl.BlockSpec((32, K1), lambda i: (0, i)),
            pl.BlockSpec((4, 32, 128), lambda i: (0, 0, 0)),
        ],
        out_specs=pl.BlockSpec((Q1, 128), lambda i: (i, 0)),
        out_shape=jax.ShapeDtypeStruct((NPAD // 4, 128), jnp.float32),
    )(tT, e)


def _sc_body(table_hbm, idx4_hbm, pe_hbm, out4_hbm,
             idx_a, idx_b, m_a, m_b, pe_v, rows_a, rows_b, tbuf_a, tbuf_b,
             sem_a, sem_b, wsem_a, wsem_b):
    wid = lax.axis_index("s") * NC + lax.axis_index("c")
    base = wid * HUPW

    pltpu.sync_copy(pe_hbm, pe_v)

    iota16 = lax.iota(jnp.int32, 16)
    off_lo = (lax.shift_left(lax.shift_right_logical(iota16, 3), 10)
              + lax.shift_left(iota16 & 7, 7))
    off_hi = off_lo + 2048

    def unpack(v):
        tl = lax.shift_right_logical(v, 6)
        tb = lax.shift_right_logical(v, 1) & 31
        h = v & 1
        return tl, tb, h

    def fetch(v, idx_v, m_v, rows_v, sem):
        tl, tb, h = unpack(v)
        pltpu.sync_copy(idx4_hbm.at[tl, tb, h], idx_v)

        def m_body(q, c):
            r = idx_v[pl.ds(q * 16, 16)]
            m = ((r & -K1)
                 | lax.shift_left(r & (Q1 - 1), 2)
                 | (lax.shift_right_logical(r, 10) & 3))
            m_v[pl.ds(q * 16, 16)] = m
            return c

        lax.fori_loop(0, HROWS // 16, m_body, 0, unroll=False)
        pltpu.async_copy(table_hbm.at[m_v], rows_v, sem)

    def compute(v, rows_v, tbuf_v, wsem):
        tl, tb, h = unpack(v)
        lbase = tl * 8 + h * 4

        def t_body(li, c_):
            l = lbase + li
            p_lo = pe_v[l, pl.ds(0, 16)]
            p_hi = pe_v[l, pl.ds(16, 16)]

            def bi_body(b8, cc):
                for u in range(8):
                    bi = b8 * 8 + u
                    row = li * 128 + bi
                    sp = jnp.zeros((16,), jnp.int32) + (li * 4096 + bi)
                    v_lo = rows_v[row, pl.ds(0, 16)] + p_lo
                    v_hi = rows_v[row, pl.ds(16, 16)] + p_hi
                    plsc.store_scatter(tbuf_v, [sp + off_lo], v_lo)
                    plsc.store_scatter(tbuf_v, [sp + off_hi], v_hi)
                return cc

            lax.fori_loop(0, 16, bi_body, 0, unroll=False)
            return c_

        lax.fori_loop(0, 4, t_body, 0, unroll=False)

        def w_body(rowid, c_):
            li = lax.shift_right_logical(rowid, 2)
            cb = rowid & 3
            l = lbase + li
            pltpu.async_copy(tbuf_v.at[pl.ds(rowid * 1024, 1024)],
                             out4_hbm.at[l, cb, tb], wsem)
            return c_

        lax.fori_loop(0, 16, w_body, 0, unroll=False)

    def drain_writes(tbuf_v, wsem):
        def wd_body(i, c_):
            pltpu.make_async_copy(tbuf_v.at[pl.ds(0, 1024)],
                                  out4_hbm.at[0, 0, 0], wsem).wait()
            return c_

        lax.fori_loop(0, 16, wd_body, 0, unroll=False)

    def drain_gather(rows_v, sem):
        pltpu.make_async_copy(table_hbm.at[pl.ds(0, HROWS)], rows_v, sem).wait()

    fetch(base, idx_a, m_a, rows_a, sem_a)

    def pair_body(n, carry):
        v0 = base + 2 * n
        v1 = v0 + 1
        vn = jnp.minimum(v0 + 2, HUNITS - 1)

        drain_gather(rows_a, sem_a)
        fetch(v1, idx_b, m_b, rows_b, sem_b)

        @pl.when(n > 0)
        def _():
            drain_writes(tbuf_a, wsem_a)

        compute(v0, rows_a, tbuf_a, wsem_a)

        drain_gather(rows_b, sem_b)
        fetch(vn, idx_a, m_a, rows_a, sem_a)

        @pl.when(n > 0)
        def _():
            drain_writes(tbuf_b, wsem_b)

        compute(v1, rows_b, tbuf_b, wsem_b)
        return carry

    lax.fori_loop(0, HUPW // 2, pair_body, 0, unroll=False)

    drain_gather(rows_a, sem_a)
    drain_writes(tbuf_a, wsem_a)
    drain_writes(tbuf_b, wsem_b)


@jax.jit
def _run(kb_ids_seq, key_emb_table):
    table_lin = _relayout_table(key_emb_table.T, jnp.asarray(_E)).reshape(NPAD, DIM)
    idx4 = kb_ids_seq.T.reshape(TL, 8, TB, 128).transpose(0, 2, 1, 3).reshape(TL, TB, 2, 512)
    mesh = plsc.VectorSubcoreMesh(core_axis_name="c", subcore_axis_name="s")
    f = pl.kernel(
        _sc_body,
        out_type=jax.ShapeDtypeStruct((L, 4, TB, 1024), jnp.float32),
        mesh=mesh,
        scratch_types=[
            pltpu.VMEM((HROWS,), jnp.int32),
            pltpu.VMEM((HROWS,), jnp.int32),
            pltpu.VMEM((HROWS,), jnp.int32),
            pltpu.VMEM((HROWS,), jnp.int32),
            pltpu.VMEM((MAX_LEN, DIM), jnp.float32),
            pltpu.VMEM((HROWS, DIM), jnp.float32),
            pltpu.VMEM((HROWS, DIM), jnp.float32),
            pltpu.VMEM((16 * 1024,), jnp.float32),
            pltpu.VMEM((16 * 1024,), jnp.float32),
            pltpu.SemaphoreType.DMA,
            pltpu.SemaphoreType.DMA,
            pltpu.SemaphoreType.DMA,
            pltpu.SemaphoreType.DMA,
        ],
        compiler_params=pltpu.CompilerParams(
            use_tc_tiling_on_sc=False, needs_layout_passes=False),
    )
    out4 = f(table_lin, idx4, jnp.asarray(_PE))
    out = (out4.reshape(L, 4, TB, 8, 128)
           .transpose(2, 4, 0, 1, 3)
           .reshape(B, L, DIM))
    return out


def kernel(kb_ids_seq, key_emb_table):
    return _run(kb_ids_seq, key_emb_table)


# speedup vs baseline: 1.2075x; 1.0035x over previous
import functools

import jax
import jax.numpy as jnp
import numpy as np
from jax import lax
from jax.experimental import pallas as pl
from jax.experimental.pallas import tpu as pltpu
from jax.experimental.pallas import tpu_sc as plsc

N_ELEMENTS = 1000000
DIM = 32
MAX_LEN = 200
B = 4096
L = 200

NC = 2
NS = 16
NW = NC * NS

K1 = 4096
Q1 = K1 // 4
NBLK1 = 245
NPAD = NBLK1 * K1

TL = L // 8
TB = B // 128
HUNITS = TL * TB * 2
HUPW = HUNITS // NW
HROWS = 4 * 128


def _sinusoidal_pe():
    pos = np.arange(MAX_LEN, dtype=np.float32)[:, None]
    div = np.exp(np.arange(0, DIM, 2, dtype=np.float32) * (-np.log(10000.0) / DIM))
    pe = np.zeros((MAX_LEN, DIM), dtype=np.float32)
    pe[:, 0::2] = np.sin(pos * div)
    pe[:, 1::2] = np.cos(pos * div)
    return pe


_PE = _sinusoidal_pe()


def _selectors():
    e = np.zeros((4, 32, 128), dtype=np.float32)
    for k in range(4):
        for c in range(32):
            e[k, c, 32 * k + c] = 1.0
    return e


_E = _selectors()


def _tc_body(x_ref, e_ref, y_ref):
    def dots(x):
        acc = jnp.zeros((Q1, 128), jnp.float32)
        for k in range(4):
            xk = x[:, k * Q1:(k + 1) * Q1]
            acc = acc + lax.dot_general(
                xk, e_ref[k], (((0,), (0,)), ((), ())),
                preferred_element_type=jnp.float32)
        return acc

    pid = pl.program_id(0)

    @pl.when(pid != NBLK1 - 1)
    def _():
        y_ref[...] = dots(x_ref[...])

    @pl.when(pid == NBLK1 - 1)
    def _():
        gcol = pid * K1 + lax.broadcasted_iota(jnp.int32, (32, K1), 1)
        y_ref[...] = dots(jnp.where(gcol < N_ELEMENTS, x_ref[...], 0.0))


def _relayout_table(tT, e):
    return pl.pallas_call(
        _tc_body,
        grid=(NBLK1,),
        in_specs=[
            pl.BlockSpec((32, K1), lambda i: (0, i)),
            pl.BlockSpec((4, 32, 128), lambda i: (0, 0, 0)),
        ],
        out_specs=pl.BlockSpec((Q1, 128), lambda i: (i, 0)),
        out_shape=jax.ShapeDtypeStruct((NPAD // 4, 128), jnp.float32),
    )(tT, e)


def _sc_body(table_hbm, idx4_hbm, pe_hbm, out4_hbm,
             idx_a, idx_b, m_a, m_b, pe_v, rows_a, rows_b, tbuf_a, tbuf_b,
             sem_a, sem_b, wsem_a, wsem_b):
    wid = lax.axis_index("s") * NC + lax.axis_index("c")
    base = wid * HUPW

    pltpu.sync_copy(pe_hbm, pe_v)

    iota16 = lax.iota(jnp.int32, 16)
    off_lo = (lax.shift_left(lax.shift_right_logical(iota16, 3), 10)
              + lax.shift_left(iota16 & 7, 7))
    off_hi = off_lo + 2048

    def unpack(v):
        tl = lax.shift_right_logical(v, 6)
        tb = lax.shift_right_logical(v, 1) & 31
        h = v & 1
        return tl, tb, h

    def fetch(v, idx_v, m_v, rows_v, sem):
        tl, tb, h = unpack(v)
        pltpu.sync_copy(idx4_hbm.at[tl, tb, h], idx_v)

        def m_body(q, c):
            r = idx_v[pl.ds(q * 16, 16)]
            m = ((r & -K1)
                 | lax.shift_left(r & (Q1 - 1), 2)
                 | (lax.shift_right_logical(r, 10) & 3))
            m_v[pl.ds(q * 16, 16)] = m
            return c

        lax.fori_loop(0, HROWS // 16, m_body, 0, unroll=False)
        pltpu.async_copy(table_hbm.at[m_v], rows_v, sem)

    def compute(v, rows_v, tbuf_v, wsem):
        tl, tb, h = unpack(v)
        lbase = tl * 8 + h * 4

        def t_body(li, c_):
            l = lbase + li
            p_lo = pe_v[l, pl.ds(0, 16)]
            p_hi = pe_v[l, pl.ds(16, 16)]
            base_lo = off_lo + li * 4096
            base_hi = base_lo + 2048

            def bi_body(bi, carry):
                s_lo, s_hi = carry
                row = li * 128 + bi
                v_lo = rows_v[row, pl.ds(0, 16)] + p_lo
                v_hi = rows_v[row, pl.ds(16, 16)] + p_hi
                plsc.store_scatter(tbuf_v, [s_lo], v_lo)
                plsc.store_scatter(tbuf_v, [s_hi], v_hi)
                return (s_lo + 1, s_hi + 1)

            lax.fori_loop(0, 128, bi_body, (base_lo, base_hi), unroll=8)
            return c_

        lax.fori_loop(0, 4, t_body, 0, unroll=False)

        def w_body(rowid, c_):
            li = lax.shift_right_logical(rowid, 2)
            cb = rowid & 3
            l = lbase + li
            pltpu.async_copy(tbuf_v.at[pl.ds(rowid * 1024, 1024)],
                             out4_hbm.at[l, cb, tb], wsem)
            return c_

        lax.fori_loop(0, 16, w_body, 0, unroll=False)

    def drain_writes(tbuf_v, wsem):
        def wd_body(i, c_):
            pltpu.make_async_copy(tbuf_v.at[pl.ds(0, 1024)],
                                  out4_hbm.at[0, 0, 0], wsem).wait()
            return c_

        lax.fori_loop(0, 16, wd_body, 0, unroll=False)

    def drain_gather(rows_v, sem):
        pltpu.make_async_copy(table_hbm.at[pl.ds(0, HROWS)], rows_v, sem).wait()

    fetch(base, idx_a, m_a, rows_a, sem_a)

    def pair_body(n, carry):
        v0 = base + 2 * n
        v1 = v0 + 1
        vn = jnp.minimum(v0 + 2, HUNITS - 1)

        drain_gather(rows_a, sem_a)
        fetch(v1, idx_b, m_b, rows_b, sem_b)

        @pl.when(n > 0)
        def _():
            drain_writes(tbuf_a, wsem_a)

        compute(v0, rows_a, tbuf_a, wsem_a)

        drain_gather(rows_b, sem_b)
        fetch(vn, idx_a, m_a, rows_a, sem_a)

        @pl.when(n > 0)
        def _():
            drain_writes(tbuf_b, wsem_b)

        compute(v1, rows_b, tbuf_b, wsem_b)
        return carry

    lax.fori_loop(0, HUPW // 2, pair_body, 0, unroll=False)

    drain_gather(rows_a, sem_a)
    drain_writes(tbuf_a, wsem_a)
    drain_writes(tbuf_b, wsem_b)


@jax.jit
def _run(kb_ids_seq, key_emb_table):
    table_lin = _relayout_table(key_emb_table.T, jnp.asarray(_E)).reshape(NPAD, DIM)
    idx4 = kb_ids_seq.T.reshape(TL, 8, TB, 128).transpose(0, 2, 1, 3).reshape(TL, TB, 2, 512)
    mesh = plsc.VectorSubcoreMesh(core_axis_name="c", subcore_axis_name="s")
    f = pl.kernel(
        _sc_body,
        out_type=jax.ShapeDtypeStruct((L, 4, TB, 1024), jnp.float32),
        mesh=mesh,
        scratch_types=[
            pltpu.VMEM((HROWS,), jnp.int32),
            pltpu.VMEM((HROWS,), jnp.int32),
            pltpu.VMEM((HROWS,), jnp.int32),
            pltpu.VMEM((HROWS,), jnp.int32),
            pltpu.VMEM((MAX_LEN, DIM), jnp.float32),
            pltpu.VMEM((HROWS, DIM), jnp.float32),
            pltpu.VMEM((HROWS, DIM), jnp.float32),
            pltpu.VMEM((16 * 1024,), jnp.float32),
            pltpu.VMEM((16 * 1024,), jnp.float32),
            pltpu.SemaphoreType.DMA,
            pltpu.SemaphoreType.DMA,
            pltpu.SemaphoreType.DMA,
            pltpu.SemaphoreType.DMA,
        ],
        compiler_params=pltpu.CompilerParams(
            use_tc_tiling_on_sc=False, needs_layout_passes=False),
    )
    out4 = f(table_lin, idx4, jnp.asarray(_PE))
    out = (out4.reshape(L, 4, TB, 8, 128)
           .transpose(2, 4, 0, 1, 3)
           .reshape(B, L, DIM))
    return out


def kernel(kb_ids_seq, key_emb_table):
    return _run(kb_ids_seq, key_emb_table)


# speedup vs baseline: 2.0893x; 1.7302x over previous
import functools

import jax
import jax.numpy as jnp
import numpy as np
from jax import lax
from jax.experimental import pallas as pl
from jax.experimental.pallas import tpu as pltpu
from jax.experimental.pallas import tpu_sc as plsc

N_ELEMENTS = 1000000
DIM = 32
MAX_LEN = 200
B = 4096
L = 200

NC = 2
NS = 16
NW = NC * NS

K1 = 4096
Q1 = K1 // 4
NBLK1 = 245
NPAD = NBLK1 * K1

TL = L // 8
TB = B // 128
HUNITS = TL * TB * 2
HUPW = HUNITS // NW
HROWS = 4 * 128


def _sinusoidal_pe():
    pos = np.arange(MAX_LEN, dtype=np.float32)[:, None]
    div = np.exp(np.arange(0, DIM, 2, dtype=np.float32) * (-np.log(10000.0) / DIM))
    pe = np.zeros((MAX_LEN, DIM), dtype=np.float32)
    pe[:, 0::2] = np.sin(pos * div)
    pe[:, 1::2] = np.cos(pos * div)
    return pe


_PE = _sinusoidal_pe()


def _selectors():
    e = np.zeros((4, 32, 128), dtype=np.float32)
    for k in range(4):
        for c in range(32):
            e[k, c, 32 * k + c] = 1.0
    return e


_E = _selectors()


def _tc_body(x_ref, e_ref, y_ref):
    def dots(x):
        acc = jnp.zeros((Q1, 128), jnp.float32)
        for k in range(4):
            xk = x[:, k * Q1:(k + 1) * Q1]
            acc = acc + lax.dot_general(
                xk, e_ref[k], (((0,), (0,)), ((), ())),
                preferred_element_type=jnp.float32)
        return acc

    pid = pl.program_id(0)

    @pl.when(pid != NBLK1 - 1)
    def _():
        y_ref[...] = dots(x_ref[...])

    @pl.when(pid == NBLK1 - 1)
    def _():
        gcol = pid * K1 + lax.broadcasted_iota(jnp.int32, (32, K1), 1)
        y_ref[...] = dots(jnp.where(gcol < N_ELEMENTS, x_ref[...], 0.0))


def _relayout_table(tT, e):
    return pl.pallas_call(
        _tc_body,
        grid=(NBLK1,),
        in_specs=[
            pl.BlockSpec((32, K1), lambda i: (0, i)),
            pl.BlockSpec((4, 32, 128), lambda i: (0, 0, 0)),
        ],
        out_specs=pl.BlockSpec((Q1, 128), lambda i: (i, 0)),
        out_shape=jax.ShapeDtypeStruct((NPAD // 4, 128), jnp.float32),
    )(tT, e)


def _sc_body(table_hbm, idx4_hbm, pe_hbm, out4_hbm,
             idx_a, idx_b, m_a, m_b, pe_v, rows_a, rows_b, tbuf_a, tbuf_b,
             sem_a, sem_b, wsem_a, wsem_b):
    wid = lax.axis_index("s") * NC + lax.axis_index("c")
    base = wid * HUPW

    pltpu.sync_copy(pe_hbm, pe_v)

    iota16 = lax.iota(jnp.int32, 16)

    def unpack(v):
        tl = lax.shift_right_logical(v, 6)
        tb = lax.shift_right_logical(v, 1) & 31
        h = v & 1
        return tl, tb, h

    def fetch(v, idx_v, m_v, rows_v, sem):
        tl, tb, h = unpack(v)
        pltpu.sync_copy(idx4_hbm.at[tl, tb, h], idx_v)

        def m_body(q, c):
            r = idx_v[pl.ds(q * 16, 16)]
            m = ((r & -K1)
                 | lax.shift_left(r & (Q1 - 1), 2)
                 | (lax.shift_right_logical(r, 10) & 3))
            m_v[pl.ds(q * 16, 16)] = m
            return c

        lax.fori_loop(0, HROWS // 16, m_body, 0, unroll=False)
        pltpu.async_copy(table_hbm.at[m_v], rows_v, sem)

    def compute(v, rows_v, tbuf_v, wsem):
        tl, tb, h = unpack(v)
        lbase = tl * 8 + h * 4

        def t_body(li, c_):
            l = lbase + li
            p_lo = pe_v[l, pl.ds(0, 16)]
            p_hi = pe_v[l, pl.ds(16, 16)]
            r_lo = iota16 + li * 32
            r_hi = r_lo + 16

            def bi_body(bi, col):
                row = li * 128 + bi
                v_lo = rows_v[row, pl.ds(0, 16)] + p_lo
                v_hi = rows_v[row, pl.ds(16, 16)] + p_hi
                plsc.store_scatter(tbuf_v, [r_lo, col], v_lo)
                plsc.store_scatter(tbuf_v, [r_hi, col], v_hi)
                return col + 1

            lax.fori_loop(0, 128, bi_body, jnp.zeros((16,), jnp.int32),
                          unroll=8)
            return c_

        lax.fori_loop(0, 4, t_body, 0, unroll=False)

        def w_body(rowid, c_):
            li = lax.shift_right_logical(rowid, 2)
            cb = rowid & 3
            l = lbase + li
            pltpu.async_copy(tbuf_v.at[pl.ds(rowid * 8, 8), pl.ds(0, 128)],
                             out4_hbm.at[l, cb, tb], wsem)
            return c_

        lax.fori_loop(0, 16, w_body, 0, unroll=False)

    def drain_writes(tbuf_v, wsem):
        def wd_body(i, c_):
            pltpu.make_async_copy(tbuf_v.at[pl.ds(0, 8), pl.ds(0, 128)],
                                  out4_hbm.at[0, 0, 0], wsem).wait()
            return c_

        lax.fori_loop(0, 16, wd_body, 0, unroll=False)

    def drain_gather(rows_v, sem):
        pltpu.make_async_copy(table_hbm.at[pl.ds(0, HROWS)], rows_v, sem).wait()

    fetch(base, idx_a, m_a, rows_a, sem_a)

    def pair_body(n, carry):
        v0 = base + 2 * n
        v1 = v0 + 1
        vn = jnp.minimum(v0 + 2, HUNITS - 1)

        drain_gather(rows_a, sem_a)
        fetch(v1, idx_b, m_b, rows_b, sem_b)

        @pl.when(n > 0)
        def _():
            drain_writes(tbuf_a, wsem_a)

        compute(v0, rows_a, tbuf_a, wsem_a)

        drain_gather(rows_b, sem_b)
        fetch(vn, idx_a, m_a, rows_a, sem_a)

        @pl.when(n > 0)
        def _():
            drain_writes(tbuf_b, wsem_b)

        compute(v1, rows_b, tbuf_b, wsem_b)
        return carry

    lax.fori_loop(0, HUPW // 2, pair_body, 0, unroll=False)

    drain_gather(rows_a, sem_a)
    drain_writes(tbuf_a, wsem_a)
    drain_writes(tbuf_b, wsem_b)


@jax.jit
def _run(kb_ids_seq, key_emb_table):
    table_lin = _relayout_table(key_emb_table.T, jnp.asarray(_E)).reshape(NPAD, DIM)
    idx4 = kb_ids_seq.T.reshape(TL, 8, TB, 128).transpose(0, 2, 1, 3).reshape(TL, TB, 2, 512)
    mesh = plsc.VectorSubcoreMesh(core_axis_name="c", subcore_axis_name="s")
    f = pl.kernel(
        _sc_body,
        out_type=jax.ShapeDtypeStruct((L, 4, TB, 8, 128), jnp.float32),
        mesh=mesh,
        scratch_types=[
            pltpu.VMEM((HROWS,), jnp.int32),
            pltpu.VMEM((HROWS,), jnp.int32),
            pltpu.VMEM((HROWS,), jnp.int32),
            pltpu.VMEM((HROWS,), jnp.int32),
            pltpu.VMEM((MAX_LEN, DIM), jnp.float32),
            pltpu.VMEM((HROWS, DIM), jnp.float32),
            pltpu.VMEM((HROWS, DIM), jnp.float32),
            pltpu.VMEM((128, 129), jnp.float32),
            pltpu.VMEM((128, 129), jnp.float32),
            pltpu.SemaphoreType.DMA,
            pltpu.SemaphoreType.DMA,
            pltpu.SemaphoreType.DMA,
            pltpu.SemaphoreType.DMA,
        ],
        compiler_params=pltpu.CompilerParams(
            use_tc_tiling_on_sc=False, needs_layout_passes=False),
    )
    out4 = f(table_lin, idx4, jnp.asarray(_PE))
    out = (out4.transpose(2, 4, 0, 1, 3)
           .reshape(B, L, DIM))
    return out


def kernel(kb_ids_seq, key_emb_table):
    return _run(kb_ids_seq, key_emb_table)


# speedup vs baseline: 2.4241x; 1.1603x over previous
import functools

import jax
import jax.numpy as jnp
import numpy as np
from jax import lax
from jax.experimental import pallas as pl
from jax.experimental.pallas import tpu as pltpu
from jax.experimental.pallas import tpu_sc as plsc

N_ELEMENTS = 1000000
DIM = 32
MAX_LEN = 200
B = 4096
L = 200

NC = 2
NS = 16
NW = NC * NS

K1 = 8192
Q1 = K1 // 4
SHQ = Q1.bit_length() - 1
NBLK1 = -(-N_ELEMENTS // K1)
NPAD = NBLK1 * K1

TL = L // 8
TB = B // 128
HUNITS = TL * TB * 2
HUPW = HUNITS // NW
HROWS = 4 * 128


def _sinusoidal_pe():
    pos = np.arange(MAX_LEN, dtype=np.float32)[:, None]
    div = np.exp(np.arange(0, DIM, 2, dtype=np.float32) * (-np.log(10000.0) / DIM))
    pe = np.zeros((MAX_LEN, DIM), dtype=np.float32)
    pe[:, 0::2] = np.sin(pos * div)
    pe[:, 1::2] = np.cos(pos * div)
    return pe


_PE = _sinusoidal_pe()


def _selectors():
    e = np.zeros((4, 32, 128), dtype=np.float32)
    for k in range(4):
        for c in range(32):
            e[k, c, 32 * k + c] = 1.0
    return e


_E = _selectors()


def _tc_body(x_ref, e_ref, y_ref):
    def dots(x):
        acc = jnp.zeros((Q1, 128), jnp.float32)
        for k in range(4):
            xk = x[:, k * Q1:(k + 1) * Q1]
            acc = acc + lax.dot_general(
                xk, e_ref[k], (((0,), (0,)), ((), ())),
                preferred_element_type=jnp.float32)
        return acc

    pid = pl.program_id(0)

    @pl.when(pid != NBLK1 - 1)
    def _():
        y_ref[...] = dots(x_ref[...])

    @pl.when(pid == NBLK1 - 1)
    def _():
        gcol = pid * K1 + lax.broadcasted_iota(jnp.int32, (32, K1), 1)
        y_ref[...] = dots(jnp.where(gcol < N_ELEMENTS, x_ref[...], 0.0))


def _relayout_table(tT, e):
    return pl.pallas_call(
        _tc_body,
        grid=(NBLK1,),
        in_specs=[
            pl.BlockSpec((32, K1), lambda i: (0, i)),
            pl.BlockSpec((4, 32, 128), lambda i: (0, 0, 0)),
        ],
        out_specs=pl.BlockSpec((Q1, 128), lambda i: (i, 0)),
        out_shape=jax.ShapeDtypeStruct((NPAD // 4, 128), jnp.float32),
    )(tT, e)


def _sc_body(table_hbm, idx4_hbm, pe_hbm, out4_hbm,
             idx_a, idx_b, m_a, m_b, pe_v, rows_a, rows_b, tbuf_a, tbuf_b,
             sem_a, sem_b, wsem_a, wsem_b):
    wid = lax.axis_index("s") * NC + lax.axis_index("c")
    base = wid * HUPW

    pltpu.sync_copy(pe_hbm, pe_v)

    iota16 = lax.iota(jnp.int32, 16)

    def unpack(v):
        tl = lax.shift_right_logical(v, 6)
        tb = lax.shift_right_logical(v, 1) & 31
        h = v & 1
        return tl, tb, h

    def fetch(v, idx_v, m_v, rows_v, sem):
        tl, tb, h = unpack(v)
        pltpu.sync_copy(idx4_hbm.at[tl, tb, h], idx_v)

        def m_body(q, c):
            r = idx_v[pl.ds(q * 16, 16)]
            m = ((r & -K1)
                 | lax.shift_left(r & (Q1 - 1), 2)
                 | (lax.shift_right_logical(r, SHQ) & 3))
            m_v[pl.ds(q * 16, 16)] = m
            return c

        lax.fori_loop(0, HROWS // 16, m_body, 0, unroll=False)
        pltpu.async_copy(table_hbm.at[m_v], rows_v, sem)

    def compute(v, rows_v, tbuf_v, wsem):
        tl, tb, h = unpack(v)
        lbase = tl * 8 + h * 4

        def t_body(li, c_):
            l = lbase + li
            p_lo = pe_v[l, pl.ds(0, 16)]
            p_hi = pe_v[l, pl.ds(16, 16)]
            r_lo = iota16 + li * 32
            r_hi = r_lo + 16

            def bi_body(bi, col):
                row = li * 128 + bi
                v_lo = rows_v[row, pl.ds(0, 16)] + p_lo
                v_hi = rows_v[row, pl.ds(16, 16)] + p_hi
                plsc.store_scatter(tbuf_v, [r_lo, col], v_lo)
                plsc.store_scatter(tbuf_v, [r_hi, col], v_hi)
                return col + 1

            lax.fori_loop(0, 128, bi_body, jnp.zeros((16,), jnp.int32),
                          unroll=8)
            return c_

        lax.fori_loop(0, 4, t_body, 0, unroll=False)

        def w_body(rowid, c_):
            li = lax.shift_right_logical(rowid, 2)
            cb = rowid & 3
            l = lbase + li
            pltpu.async_copy(tbuf_v.at[pl.ds(rowid * 8, 8), pl.ds(0, 128)],
                             out4_hbm.at[l, cb, tb], wsem)
            return c_

        lax.fori_loop(0, 16, w_body, 0, unroll=False)

    def drain_writes(tbuf_v, wsem):
        def wd_body(i, c_):
            pltpu.make_async_copy(tbuf_v.at[pl.ds(0, 8), pl.ds(0, 128)],
                                  out4_hbm.at[0, 0, 0], wsem).wait()
            return c_

        lax.fori_loop(0, 16, wd_body, 0, unroll=False)

    def drain_gather(rows_v, sem):
        pltpu.make_async_copy(table_hbm.at[pl.ds(0, HROWS)], rows_v, sem).wait()

    fetch(base, idx_a, m_a, rows_a, sem_a)

    def pair_body(n, carry):
        v0 = base + 2 * n
        v1 = v0 + 1
        vn = jnp.minimum(v0 + 2, HUNITS - 1)

        drain_gather(rows_a, sem_a)
        fetch(v1, idx_b, m_b, rows_b, sem_b)

        @pl.when(n > 0)
        def _():
            drain_writes(tbuf_a, wsem_a)

        compute(v0, rows_a, tbuf_a, wsem_a)

        drain_gather(rows_b, sem_b)
        fetch(vn, idx_a, m_a, rows_a, sem_a)

        @pl.when(n > 0)
        def _():
            drain_writes(tbuf_b, wsem_b)

        compute(v1, rows_b, tbuf_b, wsem_b)
        return carry

    lax.fori_loop(0, HUPW // 2, pair_body, 0, unroll=False)

    drain_gather(rows_a, sem_a)
    drain_writes(tbuf_a, wsem_a)
    drain_writes(tbuf_b, wsem_b)


@jax.jit
def _run(kb_ids_seq, key_emb_table):
    table_lin = _relayout_table(key_emb_table.T, jnp.asarray(_E)).reshape(NPAD, DIM)
    idx4 = kb_ids_seq.T.reshape(TL, 8, TB, 128).transpose(0, 2, 1, 3).reshape(TL, TB, 2, 512)
    mesh = plsc.VectorSubcoreMesh(core_axis_name="c", subcore_axis_name="s")
    f = pl.kernel(
        _sc_body,
        out_type=jax.ShapeDtypeStruct((L, 4, TB, 8, 128), jnp.float32),
        mesh=mesh,
        scratch_types=[
            pltpu.VMEM((HROWS,), jnp.int32),
            pltpu.VMEM((HROWS,), jnp.int32),
            pltpu.VMEM((HROWS,), jnp.int32),
            pltpu.VMEM((HROWS,), jnp.int32),
            pltpu.VMEM((MAX_LEN, DIM), jnp.float32),
            pltpu.VMEM((HROWS, DIM), jnp.float32),
            pltpu.VMEM((HROWS, DIM), jnp.float32),
            pltpu.VMEM((128, 129), jnp.float32),
            pltpu.VMEM((128, 129), jnp.float32),
            pltpu.SemaphoreType.DMA,
            pltpu.SemaphoreType.DMA,
            pltpu.SemaphoreType.DMA,
            pltpu.SemaphoreType.DMA,
        ],
        compiler_params=pltpu.CompilerParams(
            use_tc_tiling_on_sc=False, needs_layout_passes=False),
    )
    out4 = f(table_lin, idx4, jnp.asarray(_PE))
    out = (out4.transpose(2, 4, 0, 1, 3)
           .reshape(B, L, DIM))
    return out


def kernel(kb_ids_seq, key_emb_table):
    return _run(kb_ids_seq, key_emb_table)


# speedup vs baseline: 2.6464x; 1.0917x over previous
import functools

import jax
import jax.numpy as jnp
import numpy as np
from jax import lax
from jax.experimental import pallas as pl
from jax.experimental.pallas import tpu as pltpu
from jax.experimental.pallas import tpu_sc as plsc

N_ELEMENTS = 1000000
DIM = 32
MAX_LEN = 200
B = 4096
L = 200

NC = 2
NS = 16
NW = NC * NS

K1 = 16384
Q1 = K1 // 4
SHQ = Q1.bit_length() - 1
NBLK1 = -(-N_ELEMENTS // K1)
NPAD = NBLK1 * K1

TL = L // 8
TB = B // 128
HUNITS = TL * TB * 2
HUPW = HUNITS // NW
HROWS = 4 * 128


def _sinusoidal_pe():
    pos = np.arange(MAX_LEN, dtype=np.float32)[:, None]
    div = np.exp(np.arange(0, DIM, 2, dtype=np.float32) * (-np.log(10000.0) / DIM))
    pe = np.zeros((MAX_LEN, DIM), dtype=np.float32)
    pe[:, 0::2] = np.sin(pos * div)
    pe[:, 1::2] = np.cos(pos * div)
    return pe


_PE = _sinusoidal_pe()


def _selectors():
    e = np.zeros((4, 32, 128), dtype=np.float32)
    for k in range(4):
        for c in range(32):
            e[k, c, 32 * k + c] = 1.0
    return e


_E = _selectors()


def _tc_body(x_ref, e_ref, y_ref):
    def dots(x):
        acc = jnp.zeros((Q1, 128), jnp.float32)
        for k in range(4):
            xk = x[:, k * Q1:(k + 1) * Q1]
            acc = acc + lax.dot_general(
                xk, e_ref[k], (((0,), (0,)), ((), ())),
                preferred_element_type=jnp.float32)
        return acc

    pid = pl.program_id(0)

    @pl.when(pid != NBLK1 - 1)
    def _():
        y_ref[...] = dots(x_ref[...])

    @pl.when(pid == NBLK1 - 1)
    def _():
        gcol = pid * K1 + lax.broadcasted_iota(jnp.int32, (32, K1), 1)
        y_ref[...] = dots(jnp.where(gcol < N_ELEMENTS, x_ref[...], 0.0))


def _relayout_table(tT, e):
    return pl.pallas_call(
        _tc_body,
        grid=(NBLK1,),
        in_specs=[
            pl.BlockSpec((32, K1), lambda i: (0, i)),
            pl.BlockSpec((4, 32, 128), lambda i: (0, 0, 0)),
        ],
        out_specs=pl.BlockSpec((Q1, 128), lambda i: (i, 0)),
        out_shape=jax.ShapeDtypeStruct((NPAD // 4, 128), jnp.float32),
    )(tT, e)


def _sc_body(table_hbm, idx4_hbm, pe_hbm, out4_hbm,
             idx_a, idx_b, m_a, m_b, pe_v, rows_a, rows_b, tbuf_a, tbuf_b,
             sem_a, sem_b, wsem_a, wsem_b):
    wid = lax.axis_index("s") * NC + lax.axis_index("c")
    base = wid * HUPW

    pltpu.sync_copy(pe_hbm, pe_v)

    iota16 = lax.iota(jnp.int32, 16)

    def unpack(v):
        tl = lax.shift_right_logical(v, 6)
        tb = lax.shift_right_logical(v, 1) & 31
        h = v & 1
        return tl, tb, h

    def fetch(v, idx_v, m_v, rows_v, sem):
        tl, tb, h = unpack(v)
        pltpu.sync_copy(idx4_hbm.at[tl, tb, h], idx_v)

        def m_body(q, c):
            r = idx_v[pl.ds(q * 16, 16)]
            m = ((r & -K1)
                 | lax.shift_left(r & (Q1 - 1), 2)
                 | (lax.shift_right_logical(r, SHQ) & 3))
            m_v[pl.ds(q * 16, 16)] = m
            return c

        lax.fori_loop(0, HROWS // 16, m_body, 0, unroll=False)
        pltpu.async_copy(table_hbm.at[m_v], rows_v, sem)

    def compute(v, rows_v, tbuf_v, wsem):
        tl, tb, h = unpack(v)
        lbase = tl * 8 + h * 4

        def t_body(li, c_):
            l = lbase + li
            p_lo = pe_v[l, pl.ds(0, 16)]
            p_hi = pe_v[l, pl.ds(16, 16)]
            r_lo = iota16 + li * 32
            r_hi = r_lo + 16

            def bi_body(bi, col):
                row = li * 128 + bi
                v_lo = rows_v[row, pl.ds(0, 16)] + p_lo
                v_hi = rows_v[row, pl.ds(16, 16)] + p_hi
                plsc.store_scatter(tbuf_v, [r_lo, col], v_lo)
                plsc.store_scatter(tbuf_v, [r_hi, col], v_hi)
                return col + 1

            lax.fori_loop(0, 128, bi_body, jnp.zeros((16,), jnp.int32),
                          unroll=8)
            return c_

        lax.fori_loop(0, 4, t_body, 0, unroll=False)

        def w_body(rowid, c_):
            li = lax.shift_right_logical(rowid, 2)
            cb = rowid & 3
            l = lbase + li
            pltpu.async_copy(tbuf_v.at[pl.ds(rowid * 8, 8), pl.ds(0, 128)],
                             out4_hbm.at[l, cb, tb], wsem)
            return c_

        lax.fori_loop(0, 16, w_body, 0, unroll=False)

    def drain_writes(tbuf_v, wsem):
        def wd_body(i, c_):
            pltpu.make_async_copy(tbuf_v.at[pl.ds(0, 8), pl.ds(0, 128)],
                                  out4_hbm.at[0, 0, 0], wsem).wait()
            return c_

        lax.fori_loop(0, 16, wd_body, 0, unroll=False)

    def drain_gather(rows_v, sem):
        pltpu.make_async_copy(table_hbm.at[pl.ds(0, HROWS)], rows_v, sem).wait()

    fetch(base, idx_a, m_a, rows_a, sem_a)

    def pair_body(n, carry):
        v0 = base + 2 * n
        v1 = v0 + 1
        vn = jnp.minimum(v0 + 2, HUNITS - 1)

        drain_gather(rows_a, sem_a)
        fetch(v1, idx_b, m_b, rows_b, sem_b)

        @pl.when(n > 0)
        def _():
            drain_writes(tbuf_a, wsem_a)

        compute(v0, rows_a, tbuf_a, wsem_a)

        drain_gather(rows_b, sem_b)
        fetch(vn, idx_a, m_a, rows_a, sem_a)

        @pl.when(n > 0)
        def _():
            drain_writes(tbuf_b, wsem_b)

        compute(v1, rows_b, tbuf_b, wsem_b)
        return carry

    lax.fori_loop(0, HUPW // 2, pair_body, 0, unroll=False)

    drain_gather(rows_a, sem_a)
    drain_writes(tbuf_a, wsem_a)
    drain_writes(tbuf_b, wsem_b)


@jax.jit
def _run(kb_ids_seq, key_emb_table):
    table_lin = _relayout_table(key_emb_table.T, jnp.asarray(_E)).reshape(NPAD, DIM)
    idx4 = kb_ids_seq.T.reshape(TL, 8, TB, 128).transpose(0, 2, 1, 3).reshape(TL, TB, 2, 512)
    mesh = plsc.VectorSubcoreMesh(core_axis_name="c", subcore_axis_name="s")
    f = pl.kernel(
        _sc_body,
        out_type=jax.ShapeDtypeStruct((L, 4, TB, 8, 128), jnp.float32),
        mesh=mesh,
        scratch_types=[
            pltpu.VMEM((HROWS,), jnp.int32),
            pltpu.VMEM((HROWS,), jnp.int32),
            pltpu.VMEM((HROWS,), jnp.int32),
            pltpu.VMEM((HROWS,), jnp.int32),
            pltpu.VMEM((MAX_LEN, DIM), jnp.float32),
            pltpu.VMEM((HROWS, DIM), jnp.float32),
            pltpu.VMEM((HROWS, DIM), jnp.float32),
            pltpu.VMEM((128, 129), jnp.float32),
            pltpu.VMEM((128, 129), jnp.float32),
            pltpu.SemaphoreType.DMA,
            pltpu.SemaphoreType.DMA,
            pltpu.SemaphoreType.DMA,
            pltpu.SemaphoreType.DMA,
        ],
        compiler_params=pltpu.CompilerParams(
            use_tc_tiling_on_sc=False, needs_layout_passes=False),
    )
    out4 = f(table_lin, idx4, jnp.asarray(_PE))
    out = (out4.transpose(2, 4, 0, 1, 3)
           .reshape(B, L, DIM))
    return out


def kernel(kb_ids_seq, key_emb_table):
    return _run(kb_ids_seq, key_emb_table)
